# baseline (speedup 1.0000x reference)
import jax
import jax.numpy as jnp
from jax.experimental import pallas as pl
from jax.experimental.pallas import tpu as pltpu

_ENT_CHUNKS = 4


def _copy_body(ent_in, rel_in, ent_out, rel_out, ent_sems, rel_sem):
    n = ent_in.shape[0]
    chunk = n // _ENT_CHUNKS
    copies = []
    for i in range(_ENT_CHUNKS):
        lo = i * chunk
        hi = n if i == _ENT_CHUNKS - 1 else lo + chunk
        c = pltpu.make_async_copy(
            ent_in.at[pl.ds(lo, hi - lo)],
            ent_out.at[pl.ds(lo, hi - lo)],
            ent_sems.at[i],
        )
        c.start()
        copies.append(c)
    cr = pltpu.make_async_copy(rel_in, rel_out, rel_sem)
    cr.start()
    for c in copies:
        c.wait()
    cr.wait()


def kernel(entity_weight, relation_weight):
    ent_out, rel_out = pl.pallas_call(
        _copy_body,
        in_specs=[
            pl.BlockSpec(memory_space=pl.ANY),
            pl.BlockSpec(memory_space=pl.ANY),
        ],
        out_specs=[
            pl.BlockSpec(memory_space=pl.ANY),
            pl.BlockSpec(memory_space=pl.ANY),
        ],
        out_shape=[
            jax.ShapeDtypeStruct(entity_weight.shape, entity_weight.dtype),
            jax.ShapeDtypeStruct(relation_weight.shape, relation_weight.dtype),
        ],
        scratch_shapes=[
            pltpu.SemaphoreType.DMA((_ENT_CHUNKS,)),
            pltpu.SemaphoreType.DMA,
        ],
    )(entity_weight, relation_weight)
    return (ent_out, rel_out)


# speedup vs baseline: 46.2866x; 46.2866x over previous
import jax
import jax.numpy as jnp
from jax.experimental import pallas as pl
from jax.experimental.pallas import tpu as pltpu

_ENT_BLOCK = 10000


def _copy_body(ent_in, rel_in, ent_out, rel_out):
    ent_out[...] = ent_in[...]

    @pl.when(pl.program_id(0) == 0)
    def _():
        rel_out[...] = rel_in[...]


def kernel(entity_weight, relation_weight):
    n_ent, d = entity_weight.shape
    n_rel, _ = relation_weight.shape
    grid = n_ent // _ENT_BLOCK
    ent_out, rel_out = pl.pallas_call(
        _copy_body,
        grid=(grid,),
        in_specs=[
            pl.BlockSpec((_ENT_BLOCK, d), lambda i: (i, 0)),
            pl.BlockSpec((n_rel, d), lambda i: (0, 0)),
        ],
        out_specs=[
            pl.BlockSpec((_ENT_BLOCK, d), lambda i: (i, 0)),
            pl.BlockSpec((n_rel, d), lambda i: (0, 0)),
        ],
        out_shape=[
            jax.ShapeDtypeStruct(entity_weight.shape, entity_weight.dtype),
            jax.ShapeDtypeStruct(relation_weight.shape, relation_weight.dtype),
        ],
    )(entity_weight, relation_weight)
    return (ent_out, rel_out)


# speedup vs baseline: 48.5123x; 1.0481x over previous
import jax
import jax.numpy as jnp
from jax.experimental import pallas as pl
from jax.experimental.pallas import tpu as pltpu

_ENT_BLOCK = 20000


def _copy_body(ent_in, rel_in, ent_out, rel_out):
    ent_out[...] = ent_in[...]

    @pl.when(pl.program_id(0) == 0)
    def _():
        rel_out[...] = rel_in[...]


def kernel(entity_weight, relation_weight):
    n_ent, d = entity_weight.shape
    n_rel, _ = relation_weight.shape
    grid = n_ent // _ENT_BLOCK
    ent_out, rel_out = pl.pallas_call(
        _copy_body,
        grid=(grid,),
        in_specs=[
            pl.BlockSpec((_ENT_BLOCK, d), lambda i: (i, 0)),
            pl.BlockSpec((n_rel, d), lambda i: (0, 0)),
        ],
        out_specs=[
            pl.BlockSpec((_ENT_BLOCK, d), lambda i: (i, 0)),
            pl.BlockSpec((n_rel, d), lambda i: (0, 0)),
        ],
        out_shape=[
            jax.ShapeDtypeStruct(entity_weight.shape, entity_weight.dtype),
            jax.ShapeDtypeStruct(relation_weight.shape, relation_weight.dtype),
        ],
    )(entity_weight, relation_weight)
    return (ent_out, rel_out)


# speedup vs baseline: 48.7081x; 1.0040x over previous
import jax
import jax.numpy as jnp
from jax.experimental import pallas as pl
from jax.experimental.pallas import tpu as pltpu

_ENT_BLOCK = 25000


def _copy_body(ent_in, rel_in, ent_out, rel_out):
    ent_out[...] = ent_in[...]

    @pl.when(pl.program_id(0) == 0)
    def _():
        rel_out[...] = rel_in[...]


def kernel(entity_weight, relation_weight):
    n_ent, d = entity_weight.shape
    n_rel, _ = relation_weight.shape
    grid = n_ent // _ENT_BLOCK
    ent_out, rel_out = pl.pallas_call(
        _copy_body,
        grid=(grid,),
        in_specs=[
            pl.BlockSpec((_ENT_BLOCK, d), lambda i: (i, 0)),
            pl.BlockSpec((n_rel, d), lambda i: (0, 0)),
        ],
        out_specs=[
            pl.BlockSpec((_ENT_BLOCK, d), lambda i: (i, 0)),
            pl.BlockSpec((n_rel, d), lambda i: (0, 0)),
        ],
        out_shape=[
            jax.ShapeDtypeStruct(entity_weight.shape, entity_weight.dtype),
            jax.ShapeDtypeStruct(relation_weight.shape, relation_weight.dtype),
        ],
    )(entity_weight, relation_weight)
    return (ent_out, rel_out)
